# baseline (speedup 1.0000x reference)
import functools

import jax
import jax.numpy as jnp
import numpy as np
from jax import lax
from jax.experimental import pallas as pl
from jax.experimental.pallas import tpu as pltpu
from jax.experimental.pallas import tpu_sc as plsc

_VOCAB = 100000
_D = 1024
_BATCH = 4
_SEQ = 2048
_N = _BATCH * _SEQ

_NC = 2
_NS = 16
_NW = _NC * _NS
_CHUNK = 32

_K = 4
_CW = _SEQ // _K
_NROWS_C = _BATCH * _CW
_BPW = _NROWS_C // _NW
_NCHUNK = _BPW // _CHUNK

_SCALE = float(np.sqrt(_D))


def _pe_table() -> np.ndarray:
    pos = np.arange(_SEQ, dtype=np.float64)[:, None]
    i = np.arange(0, _D, 2, dtype=np.float64)
    div = np.exp(-np.log(10000.0) * i / _D)
    pe = np.zeros((_SEQ, _D), dtype=np.float64)
    pe[:, 0::2] = np.sin(pos * div)
    pe[:, 1::2] = np.cos(pos * div)
    return pe.astype(np.float32)


_PE = _pe_table()


def _sc_gather(table, idx3):
    mesh = plsc.VectorSubcoreMesh(core_axis_name="c", subcore_axis_name="s")

    @functools.partial(
        pl.kernel,
        mesh=mesh,
        out_type=jax.ShapeDtypeStruct((_NROWS_C, _D), jnp.float32),
        scratch_types=[
            pltpu.VMEM((_NCHUNK, _CHUNK), jnp.int32),
            pltpu.VMEM((_CHUNK, _D), jnp.float32),
            pltpu.VMEM((_CHUNK, _D), jnp.float32),
            pltpu.SemaphoreType.DMA,
            pltpu.SemaphoreType.DMA,
        ],
    )
    def k(table_hbm, idx_hbm, out_hbm, idx_v, rows0, rows1, gsem, osem):
        wid = lax.axis_index("s") * _NC + lax.axis_index("c")
        base = wid * _BPW
        pltpu.sync_copy(idx_hbm.at[wid], idx_v)

        def _wait_gather(buf):
            pltpu.make_async_copy(table_hbm.at[pl.ds(0, _CHUNK)], buf, gsem).wait()

        def _step(j, cur, nxt):
            _wait_gather(cur)

            @pl.when(j + 1 < _NCHUNK)
            def _():
                pltpu.async_copy(table_hbm.at[idx_v.at[j + 1]], nxt, gsem)

            pltpu.sync_copy(cur, out_hbm.at[pl.ds(base + j * _CHUNK, _CHUNK)])

        pltpu.async_copy(table_hbm.at[idx_v.at[0]], rows0, gsem)

        @pl.loop(0, _NCHUNK, step=2)
        def _(j):
            _step(j, rows0, rows1)
            _step(j + 1, rows1, rows0)

    return k(table, idx3)


def _fixup_chunk(prev, gathered, pe, c):
    def body(*refs):
        g_ref, p_ref, o_ref = refs[-3], refs[-2], refs[-1]
        o_ref[...] = g_ref[...] * _SCALE + p_ref[...]

    in_specs = [
        pl.BlockSpec((_CW, _D), lambda b: (b, 0)),
        pl.BlockSpec((_CW, _D), lambda b: (c, 0)),
    ]
    operands = [gathered, pe]
    aliases = {}
    if prev is not None:
        in_specs = [pl.BlockSpec(memory_space=pl.ANY)] + in_specs
        operands = [prev] + operands
        aliases = {0: 0}

    return pl.pallas_call(
        body,
        grid=(_BATCH,),
        in_specs=in_specs,
        out_specs=pl.BlockSpec((_CW, _D), lambda b: (b * _K + c, 0)),
        out_shape=jax.ShapeDtypeStruct((_N, _D), jnp.float32),
        input_output_aliases=aliases,
    )(*operands)


def kernel(tokens, table):
    pe = jnp.asarray(_PE)
    tok = tokens.astype(jnp.int32)
    out = None
    for c in range(_K):
        idx3 = tok[:, c * _CW:(c + 1) * _CW].reshape(_NW, _NCHUNK, _CHUNK)
        g = _sc_gather(table, idx3)
        out = _fixup_chunk(out, g, pe, c)
    return out.reshape(_BATCH, _SEQ, _D)


# speedup vs baseline: 1.1178x; 1.1178x over previous
import functools

import jax
import jax.numpy as jnp
import numpy as np
from jax import lax
from jax.experimental import pallas as pl
from jax.experimental.pallas import tpu as pltpu
from jax.experimental.pallas import tpu_sc as plsc

_VOCAB = 100000
_D = 1024
_BATCH = 4
_SEQ = 2048
_N = _BATCH * _SEQ

_NC = 2
_NS = 16
_NW = _NC * _NS
_CHUNK = 32

_K = 1
_CW = _SEQ // _K
_NROWS_C = _BATCH * _CW
_BPW = _NROWS_C // _NW
_NCHUNK = _BPW // _CHUNK

_SCALE = float(np.sqrt(_D))


def _pe_table() -> np.ndarray:
    pos = np.arange(_SEQ, dtype=np.float64)[:, None]
    i = np.arange(0, _D, 2, dtype=np.float64)
    div = np.exp(-np.log(10000.0) * i / _D)
    pe = np.zeros((_SEQ, _D), dtype=np.float64)
    pe[:, 0::2] = np.sin(pos * div)
    pe[:, 1::2] = np.cos(pos * div)
    return pe.astype(np.float32)


_PE = _pe_table()


def _sc_gather(table, idx3):
    mesh = plsc.VectorSubcoreMesh(core_axis_name="c", subcore_axis_name="s")

    @functools.partial(
        pl.kernel,
        mesh=mesh,
        out_type=jax.ShapeDtypeStruct((_NROWS_C, _D), jnp.float32),
        scratch_types=[
            pltpu.VMEM((_NCHUNK, _CHUNK), jnp.int32),
            pltpu.VMEM((_CHUNK, _D), jnp.float32),
            pltpu.VMEM((_CHUNK, _D), jnp.float32),
            pltpu.SemaphoreType.DMA,
            pltpu.SemaphoreType.DMA,
        ],
    )
    def k(table_hbm, idx_hbm, out_hbm, idx_v, rows0, rows1, gsem, osem):
        wid = lax.axis_index("s") * _NC + lax.axis_index("c")
        base = wid * _BPW
        pltpu.sync_copy(idx_hbm.at[wid], idx_v)

        def _wait_gather(buf):
            pltpu.make_async_copy(table_hbm.at[pl.ds(0, _CHUNK)], buf, gsem).wait()

        def _step(j, cur, nxt):
            _wait_gather(cur)

            @pl.when(j + 1 < _NCHUNK)
            def _():
                pltpu.async_copy(table_hbm.at[idx_v.at[j + 1]], nxt, gsem)

            pltpu.sync_copy(cur, out_hbm.at[pl.ds(base + j * _CHUNK, _CHUNK)])

        pltpu.async_copy(table_hbm.at[idx_v.at[0]], rows0, gsem)

        @pl.loop(0, _NCHUNK, step=2)
        def _(j):
            _step(j, rows0, rows1)
            _step(j + 1, rows1, rows0)

    return k(table, idx3)


def _fixup_chunk(prev, gathered, pe, c):
    def body(*refs):
        g_ref, p_ref, o_ref = refs[-3], refs[-2], refs[-1]
        o_ref[...] = g_ref[...] * _SCALE + p_ref[...]

    in_specs = [
        pl.BlockSpec((_CW, _D), lambda b: (b, 0)),
        pl.BlockSpec((_CW, _D), lambda b: (c, 0)),
    ]
    operands = [gathered, pe]
    aliases = {}
    if prev is not None:
        in_specs = [pl.BlockSpec(memory_space=pl.ANY)] + in_specs
        operands = [prev] + operands
        aliases = {0: 0}

    return pl.pallas_call(
        body,
        grid=(_BATCH,),
        in_specs=in_specs,
        out_specs=pl.BlockSpec((_CW, _D), lambda b: (b * _K + c, 0)),
        out_shape=jax.ShapeDtypeStruct((_N, _D), jnp.float32),
        input_output_aliases=aliases,
    )(*operands)


def kernel(tokens, table):
    pe = jnp.asarray(_PE)
    tok = tokens.astype(jnp.int32)
    out = None
    for c in range(_K):
        idx3 = tok[:, c * _CW:(c + 1) * _CW].reshape(_NW, _NCHUNK, _CHUNK)
        g = _sc_gather(table, idx3)
        out = _fixup_chunk(out, g, pe, c)
    return out.reshape(_BATCH, _SEQ, _D)
